# baseline (speedup 1.0000x reference)
import functools

import jax
import jax.numpy as jnp
from jax import lax
from jax.experimental import pallas as pl
from jax.experimental.pallas import tpu as pltpu
from jax.experimental.pallas import tpu_sc as plsc

T, D, F, E = 2048, 1024, 2048, 8
BLK = 256
CAP = 2 * T + E * BLK
NB = CAP // BLK
NBP = 32
NW = 32
TPW = T // NW


def _router_kernel(logits_ref, pos1_ref, pos2_ref, wt1_ref, be_ref, act_ref,
                   xi_ref):
    l = logits_ref[...]
    ids = lax.broadcasted_iota(jnp.int32, (T, E), 1)
    m1 = jnp.max(l, axis=1, keepdims=True)
    i1 = jnp.argmax(l, axis=1)[:, None]
    masked = jnp.where(ids == i1, -jnp.inf, l)
    m2 = jnp.max(masked, axis=1, keepdims=True)
    i2 = jnp.argmax(masked, axis=1)[:, None]
    w1 = 1.0 / (1.0 + jnp.exp(m2 - m1))
    wt1_ref[...] = jnp.broadcast_to(w1, (T, 16))

    match = ((ids == i1) | (ids == i2)).astype(jnp.bfloat16)
    r = lax.broadcasted_iota(jnp.int32, (T, T), 0)
    c = lax.broadcasted_iota(jnp.int32, (T, T), 1)
    tri = (c < r).astype(jnp.bfloat16)
    rank = lax.dot_general(tri, match, (((1,), (0,)), ((), ())),
                           preferred_element_type=jnp.float32)
    counts = jnp.sum(match.astype(jnp.float32), axis=0)
    cnt = counts.astype(jnp.int32)

    pos1 = jnp.zeros((T, 1), jnp.int32)
    pos2 = jnp.zeros((T, 1), jnp.int32)
    start = jnp.int32(0)
    starts = []
    for e in range(E):
        starts.append(start)
        start = start + ((cnt[e] + BLK - 1) // BLK) * BLK
    for e in range(E):
        pe = starts[e] + rank[:, e:e + 1].astype(jnp.int32)
        pos1 = jnp.where(i1 == e, pe, pos1)
        pos2 = jnp.where(i2 == e, pe, pos2)
    pos1_ref[...] = pos1
    pos2_ref[...] = pos2

    blk_base = lax.broadcasted_iota(jnp.int32, (NBP, 1), 0) * BLK
    be = jnp.zeros((NBP, 1), jnp.int32)
    for e in range(1, E):
        be = be + (blk_base >= starts[e]).astype(jnp.int32)
    act = (blk_base < start).astype(jnp.int32)
    be_last = jnp.max(be * act)
    be_ref[...] = jnp.where(act > 0, be, be_last)
    act_ref[...] = act
    nact = (start + BLK - 1) // BLK
    blk_i = lax.broadcasted_iota(jnp.int32, (NBP, 1), 0)
    xi_ref[...] = jnp.minimum(blk_i, jnp.maximum(nact - 1, 0))


def _router(router_logits):
    return pl.pallas_call(
        _router_kernel,
        out_shape=[
            jax.ShapeDtypeStruct((T, 1), jnp.int32),
            jax.ShapeDtypeStruct((T, 1), jnp.int32),
            jax.ShapeDtypeStruct((T, 16), jnp.float32),
            jax.ShapeDtypeStruct((NBP, 1), jnp.int32),
            jax.ShapeDtypeStruct((NBP, 1), jnp.int32),
            jax.ShapeDtypeStruct((NBP, 1), jnp.int32),
        ],
    )(router_logits)


_DCH = TPW // 2


def _dispatch_body(x_hbm, pos1_hbm, pos2_hbm, disp_hbm,
                   ra_v, rb_v, p1_v, p2_v, semL, sem0, sem1):
    wid = lax.axis_index("s") * 2 + lax.axis_index("c")
    base = wid * TPW
    pltpu.sync_copy(pos1_hbm.at[pl.ds(base, _DCH)], p1_v.at[0])
    pltpu.sync_copy(pos1_hbm.at[pl.ds(base + _DCH, _DCH)], p1_v.at[1])
    pltpu.sync_copy(pos2_hbm.at[pl.ds(base, _DCH)], p2_v.at[0])
    pltpu.sync_copy(pos2_hbm.at[pl.ds(base + _DCH, _DCH)], p2_v.at[1])
    la = pltpu.make_async_copy(x_hbm.at[pl.ds(base, _DCH)], ra_v, semL)
    la.start()
    lb = pltpu.make_async_copy(x_hbm.at[pl.ds(base + _DCH, _DCH)], rb_v, semL)
    la.wait()
    lb.start()
    s1a = pltpu.make_async_copy(ra_v, disp_hbm.at[p1_v.at[0]], sem0)
    s2a = pltpu.make_async_copy(ra_v, disp_hbm.at[p2_v.at[0]], sem1)
    s1a.start()
    s2a.start()
    lb.wait()
    s1b = pltpu.make_async_copy(rb_v, disp_hbm.at[p1_v.at[1]], sem0)
    s2b = pltpu.make_async_copy(rb_v, disp_hbm.at[p2_v.at[1]], sem1)
    s1b.start()
    s2b.start()
    s1a.wait()
    s2a.wait()
    s1b.wait()
    s2b.wait()


def _dispatch(hidden_states, pos1, pos2):
    mesh = plsc.VectorSubcoreMesh(core_axis_name="c", subcore_axis_name="s", num_cores=2, num_subcores=16)
    return pl.kernel(
        _dispatch_body,
        out_type=jax.ShapeDtypeStruct((CAP, D), jnp.float32),
        mesh=mesh,
        scratch_types=[
            pltpu.VMEM((_DCH, D), jnp.float32),
            pltpu.VMEM((_DCH, D), jnp.float32),
            pltpu.VMEM((2, _DCH), jnp.int32),
            pltpu.VMEM((2, _DCH), jnp.int32),
            pltpu.SemaphoreType.DMA,
            pltpu.SemaphoreType.DMA,
            pltpu.SemaphoreType.DMA,
        ],
    )(hidden_states, pos1, pos2)


def _gateup_kernel(be_sref, act_sref, xi_sref, x_ref, w13_ref, g_ref):
    del be_sref, xi_sref

    @pl.when(act_sref[pl.program_id(0)] > 0)
    def _():
        x = x_ref[...].astype(jnp.bfloat16)
        xT = x.T
        w13 = w13_ref[0]
        hT = lax.dot_general(w13, xT, (((1,), (0,)), ((), ())),
                             preferred_element_type=jnp.float32)
        hg = hT[:F, :]
        hu = hT[F:, :]
        g_ref[...] = (hg * jax.nn.sigmoid(hg) * hu).astype(jnp.bfloat16)


def _gateup(disp, w13, be, act, xi):
    grid_spec = pltpu.PrefetchScalarGridSpec(
        num_scalar_prefetch=3,
        grid=(NB,),
        in_specs=[
            pl.BlockSpec((BLK, D), lambda i, be, act, xi: (xi[i], 0)),
            pl.BlockSpec((1, 2 * F, D),
                         lambda i, be, act, xi: (be[i], 0, 0)),
        ],
        out_specs=pl.BlockSpec((F, BLK), lambda i, be, act, xi: (0, xi[i])),
    )
    return pl.pallas_call(
        _gateup_kernel,
        grid_spec=grid_spec,
        out_shape=jax.ShapeDtypeStruct((F, CAP), jnp.bfloat16),
    )(be, act, xi, disp, w13)


def _down_kernel(be_sref, act_sref, xi_sref, g_ref, w2_ref, y_ref):
    del be_sref, xi_sref

    @pl.when(act_sref[pl.program_id(0)] > 0)
    def _():
        g = g_ref[...]
        w2 = w2_ref[0]
        yT = lax.dot_general(w2, g, (((1,), (0,)), ((), ())),
                             preferred_element_type=jnp.float32)
        y = yT.T
        u_lo = lax.bitcast_convert_type(y[:, :D // 2], jnp.int32)
        u_hi = lax.bitcast_convert_type(y[:, D // 2:], jnp.int32)
        r_lo = u_lo + 0x7FFF + ((u_lo >> 16) & 1)
        r_hi = u_hi + 0x7FFF + ((u_hi >> 16) & 1)
        lo16 = lax.shift_right_logical(r_lo, 16)
        hi16 = r_hi & jnp.int32(-65536)
        y_ref[...] = lo16 | hi16


def _down(g, w2, be, act, xi):
    grid_spec = pltpu.PrefetchScalarGridSpec(
        num_scalar_prefetch=3,
        grid=(NB,),
        in_specs=[
            pl.BlockSpec((F, BLK), lambda i, be, act, xi: (0, xi[i])),
            pl.BlockSpec((1, D, F), lambda i, be, act, xi: (be[i], 0, 0)),
        ],
        out_specs=pl.BlockSpec((BLK, D // 2),
                               lambda i, be, act, xi: (xi[i], 0)),
    )
    return pl.pallas_call(
        _down_kernel,
        grid_spec=grid_spec,
        out_shape=jax.ShapeDtypeStruct((CAP, D // 2), jnp.int32),
    )(be, act, xi, g, w2)


_CCH = 16
_NCH = TPW // _CCH


def _combine_body(y_hbm, pos1_hbm, pos2_hbm, wt1_hbm, out_hbm,
                  r1s, r2s, os_, p1s, p2s, w1s,
                  semg0, semg1, semo0, semo1):
    wid = lax.axis_index("s") * 2 + lax.axis_index("c")
    base = wid * TPW
    himask = jnp.int32(-65536)
    semg = (semg0, semg1)
    semo = (semo0, semo1)
    pltpu.sync_copy(wt1_hbm.at[pl.ds(base, TPW)], w1s)
    for c in range(_NCH):
        pltpu.sync_copy(pos1_hbm.at[pl.ds(base + c * _CCH, _CCH)], p1s.at[c])
        pltpu.sync_copy(pos2_hbm.at[pl.ds(base + c * _CCH, _CCH)], p2s.at[c])

    def gathers(c):
        b = c % 2
        g1 = pltpu.make_async_copy(y_hbm.at[p1s.at[c]], r1s.at[b], semg[b])
        g2 = pltpu.make_async_copy(y_hbm.at[p2s.at[c]], r2s.at[b], semg[b])
        g1.start()
        g2.start()
        return g1, g2

    inflight = [gathers(0), gathers(1)]
    writes = [None, None]
    for c in range(_NCH):
        b = c % 2
        g1, g2 = inflight[b]
        g1.wait()
        g2.wait()
        if writes[b] is not None:
            writes[b].wait()

        def body(t, carry, c=c, b=b):
            w1 = w1s[c * _CCH + t, :]
            w2 = 1.0 - w1
            for j in range(D // 32):
                sl = pl.ds(j * 16, 16)
                v1 = r1s[b, t, sl]
                v2 = r2s[b, t, sl]
                lo1 = lax.bitcast_convert_type(v1 << 16, jnp.float32)
                hi1 = lax.bitcast_convert_type(v1 & himask, jnp.float32)
                lo2 = lax.bitcast_convert_type(v2 << 16, jnp.float32)
                hi2 = lax.bitcast_convert_type(v2 & himask, jnp.float32)
                os_[b, t, sl] = w1 * lo1 + w2 * lo2
                os_[b, t, pl.ds(D // 2 + j * 16, 16)] = w1 * hi1 + w2 * hi2
            return carry

        lax.fori_loop(0, _CCH, body, jnp.int32(0))
        wr = pltpu.make_async_copy(
            os_.at[b], out_hbm.at[pl.ds(base + c * _CCH, _CCH)], semo[b])
        wr.start()
        writes[b] = wr
        if c + 2 < _NCH:
            inflight[b] = gathers(c + 2)
    writes[0].wait()
    writes[1].wait()


def _combine(y, pos1, pos2, wt1):
    mesh = plsc.VectorSubcoreMesh(core_axis_name="c", subcore_axis_name="s", num_cores=2, num_subcores=16)
    return pl.kernel(
        _combine_body,
        out_type=jax.ShapeDtypeStruct((T, D), jnp.float32),
        mesh=mesh,
        scratch_types=[
            pltpu.VMEM((2, _CCH, D // 2), jnp.int32),
            pltpu.VMEM((2, _CCH, D // 2), jnp.int32),
            pltpu.VMEM((2, _CCH, D), jnp.float32),
            pltpu.VMEM((_NCH, _CCH), jnp.int32),
            pltpu.VMEM((_NCH, _CCH), jnp.int32),
            pltpu.VMEM((TPW, 16), jnp.float32),
            pltpu.SemaphoreType.DMA,
            pltpu.SemaphoreType.DMA,
            pltpu.SemaphoreType.DMA,
            pltpu.SemaphoreType.DMA,
        ],
    )(y, pos1, pos2, wt1)


@jax.jit
def kernel(hidden_states, router_logits, w13, w2):
    pos1, pos2, wt1, be, act, xi = _router(router_logits)
    pos1 = pos1.reshape(T)
    pos2 = pos2.reshape(T)
    be = be.reshape(NBP)
    act = act.reshape(NBP)
    xi = xi.reshape(NBP)
    disp = _dispatch(hidden_states, pos1, pos2)
    g = _gateup(disp, w13, be, act, xi)
    y = _down(g, w2, be, act, xi)
    return _combine(y, pos1, pos2, wt1)


# speedup vs baseline: 1.0070x; 1.0070x over previous
import functools

import jax
import jax.numpy as jnp
from jax import lax
from jax.experimental import pallas as pl
from jax.experimental.pallas import tpu as pltpu
from jax.experimental.pallas import tpu_sc as plsc

T, D, F, E = 2048, 1024, 2048, 8
BLK = 256
CAP = 2 * T + E * BLK
NB = CAP // BLK
NBP = 32
NW = 32
TPW = T // NW


def _router_kernel(logits_ref, pos1_ref, pos2_ref, wt1_ref, be_ref, act_ref,
                   xi_ref):
    l = logits_ref[...]
    ids = lax.broadcasted_iota(jnp.int32, (T, E), 1)
    m1 = jnp.max(l, axis=1, keepdims=True)
    i1 = jnp.argmax(l, axis=1)[:, None]
    masked = jnp.where(ids == i1, -jnp.inf, l)
    m2 = jnp.max(masked, axis=1, keepdims=True)
    i2 = jnp.argmax(masked, axis=1)[:, None]
    w1 = 1.0 / (1.0 + jnp.exp(m2 - m1))
    wt1_ref[...] = jnp.broadcast_to(w1, (T, 16))

    match = ((ids == i1) | (ids == i2)).astype(jnp.bfloat16)
    r = lax.broadcasted_iota(jnp.int32, (T, T), 0)
    c = lax.broadcasted_iota(jnp.int32, (T, T), 1)
    tri = (c < r).astype(jnp.bfloat16)
    rank = lax.dot_general(tri, match, (((1,), (0,)), ((), ())),
                           preferred_element_type=jnp.float32)
    counts = jnp.sum(match.astype(jnp.float32), axis=0)
    cnt = counts.astype(jnp.int32)

    pos1 = jnp.zeros((T, 1), jnp.int32)
    pos2 = jnp.zeros((T, 1), jnp.int32)
    start = jnp.int32(0)
    starts = []
    for e in range(E):
        starts.append(start)
        start = start + ((cnt[e] + BLK - 1) // BLK) * BLK
    for e in range(E):
        pe = starts[e] + rank[:, e:e + 1].astype(jnp.int32)
        pos1 = jnp.where(i1 == e, pe, pos1)
        pos2 = jnp.where(i2 == e, pe, pos2)
    pos1_ref[...] = pos1
    pos2_ref[...] = pos2

    blk_base = lax.broadcasted_iota(jnp.int32, (NBP, 1), 0) * BLK
    be = jnp.zeros((NBP, 1), jnp.int32)
    for e in range(1, E):
        be = be + (blk_base >= starts[e]).astype(jnp.int32)
    act = (blk_base < start).astype(jnp.int32)
    be_last = jnp.max(be * act)
    be_ref[...] = jnp.where(act > 0, be, be_last)
    act_ref[...] = act
    nact = (start + BLK - 1) // BLK
    blk_i = lax.broadcasted_iota(jnp.int32, (NBP, 1), 0)
    xi_ref[...] = jnp.minimum(blk_i, jnp.maximum(nact - 1, 0))


def _router(router_logits):
    return pl.pallas_call(
        _router_kernel,
        out_shape=[
            jax.ShapeDtypeStruct((T, 1), jnp.int32),
            jax.ShapeDtypeStruct((T, 1), jnp.int32),
            jax.ShapeDtypeStruct((T, 16), jnp.float32),
            jax.ShapeDtypeStruct((NBP, 1), jnp.int32),
            jax.ShapeDtypeStruct((NBP, 1), jnp.int32),
            jax.ShapeDtypeStruct((NBP, 1), jnp.int32),
        ],
    )(router_logits)


def _dispatch_body(x_hbm, pos1_hbm, pos2_hbm, disp_hbm,
                   rows_v, p1_v, p2_v, sem0, sem1):
    wid = lax.axis_index("s") * 2 + lax.axis_index("c")
    base = wid * TPW
    pltpu.sync_copy(x_hbm.at[pl.ds(base, TPW)], rows_v)
    pltpu.sync_copy(pos1_hbm.at[pl.ds(base, TPW)], p1_v)
    pltpu.sync_copy(pos2_hbm.at[pl.ds(base, TPW)], p2_v)
    c1 = pltpu.make_async_copy(rows_v, disp_hbm.at[p1_v], sem0)
    c1.start()
    c2 = pltpu.make_async_copy(rows_v, disp_hbm.at[p2_v], sem1)
    c2.start()
    c1.wait()
    c2.wait()


def _dispatch(hidden_states, pos1, pos2):
    mesh = plsc.VectorSubcoreMesh(core_axis_name="c", subcore_axis_name="s", num_cores=2, num_subcores=16)
    return pl.kernel(
        _dispatch_body,
        out_type=jax.ShapeDtypeStruct((CAP, D), jnp.float32),
        mesh=mesh,
        scratch_types=[
            pltpu.VMEM((TPW, D), jnp.float32),
            pltpu.VMEM((TPW,), jnp.int32),
            pltpu.VMEM((TPW,), jnp.int32),
            pltpu.SemaphoreType.DMA,
            pltpu.SemaphoreType.DMA,
        ],
    )(hidden_states, pos1, pos2)


def _gateup_kernel(be_sref, act_sref, xi_sref, x_ref, w13_ref, g_ref):
    del be_sref, xi_sref

    @pl.when(act_sref[pl.program_id(0)] > 0)
    def _():
        x = x_ref[...].astype(jnp.bfloat16)
        xT = x.T
        w13 = w13_ref[0]
        hT = lax.dot_general(w13, xT, (((1,), (0,)), ((), ())),
                             preferred_element_type=jnp.float32)
        hg = hT[:F, :]
        hu = hT[F:, :]
        g_ref[...] = (hg * jax.nn.sigmoid(hg) * hu).astype(jnp.bfloat16)


def _gateup(disp, w13, be, act, xi):
    grid_spec = pltpu.PrefetchScalarGridSpec(
        num_scalar_prefetch=3,
        grid=(NB,),
        in_specs=[
            pl.BlockSpec((BLK, D), lambda i, be, act, xi: (xi[i], 0)),
            pl.BlockSpec((1, 2 * F, D),
                         lambda i, be, act, xi: (be[i], 0, 0)),
        ],
        out_specs=pl.BlockSpec((F, BLK), lambda i, be, act, xi: (0, xi[i])),
    )
    return pl.pallas_call(
        _gateup_kernel,
        grid_spec=grid_spec,
        out_shape=jax.ShapeDtypeStruct((F, CAP), jnp.bfloat16),
    )(be, act, xi, disp, w13)


def _down_kernel(be_sref, act_sref, xi_sref, g_ref, w2_ref, y_ref):
    del be_sref, xi_sref

    @pl.when(act_sref[pl.program_id(0)] > 0)
    def _():
        g = g_ref[...]
        w2 = w2_ref[0]
        yT = lax.dot_general(w2, g, (((1,), (0,)), ((), ())),
                             preferred_element_type=jnp.float32)
        y_ref[...] = yT.T


def _down(g, w2, be, act, xi):
    grid_spec = pltpu.PrefetchScalarGridSpec(
        num_scalar_prefetch=3,
        grid=(NB,),
        in_specs=[
            pl.BlockSpec((F, BLK), lambda i, be, act, xi: (0, xi[i])),
            pl.BlockSpec((1, D, F), lambda i, be, act, xi: (be[i], 0, 0)),
        ],
        out_specs=pl.BlockSpec((BLK, D),
                               lambda i, be, act, xi: (xi[i], 0)),
    )
    return pl.pallas_call(
        _down_kernel,
        grid_spec=grid_spec,
        out_shape=jax.ShapeDtypeStruct((CAP, D), jnp.float32),
    )(be, act, xi, g, w2)


_CCH = 32


def _combine_body(y_hbm, pos1_hbm, pos2_hbm, wt1_hbm, out_hbm,
                  r1_v, r2_v, p1_v, p2_v, w1_v, sem, sem2):
    wid = lax.axis_index("s") * 2 + lax.axis_index("c")
    base = wid * TPW
    for chunk in range(TPW // _CCH):
        cbase = base + chunk * _CCH
        pltpu.sync_copy(pos1_hbm.at[pl.ds(cbase, _CCH)], p1_v)
        pltpu.sync_copy(pos2_hbm.at[pl.ds(cbase, _CCH)], p2_v)
        pltpu.sync_copy(wt1_hbm.at[pl.ds(cbase, _CCH)], w1_v)
        c1 = pltpu.make_async_copy(y_hbm.at[p1_v], r1_v, sem)
        c1.start()
        c2 = pltpu.make_async_copy(y_hbm.at[p2_v], r2_v, sem2)
        c2.start()
        c1.wait()
        c2.wait()

        def body(t, carry):
            w1 = w1_v[t, :]
            w2 = 1.0 - w1
            for j in range(D // 16):
                sl = pl.ds(j * 16, 16)
                r1_v[t, sl] = w1 * r1_v[t, sl] + w2 * r2_v[t, sl]
            return carry

        lax.fori_loop(0, _CCH, body, jnp.int32(0))
        pltpu.sync_copy(r1_v, out_hbm.at[pl.ds(cbase, _CCH)])


def _combine(y, pos1, pos2, wt1):
    mesh = plsc.VectorSubcoreMesh(core_axis_name="c", subcore_axis_name="s", num_cores=2, num_subcores=16)
    return pl.kernel(
        _combine_body,
        out_type=jax.ShapeDtypeStruct((T, D), jnp.float32),
        mesh=mesh,
        scratch_types=[
            pltpu.VMEM((_CCH, D), jnp.float32),
            pltpu.VMEM((_CCH, D), jnp.float32),
            pltpu.VMEM((_CCH,), jnp.int32),
            pltpu.VMEM((_CCH,), jnp.int32),
            pltpu.VMEM((_CCH, 16), jnp.float32),
            pltpu.SemaphoreType.DMA,
            pltpu.SemaphoreType.DMA,
        ],
    )(y, pos1, pos2, wt1)


@jax.jit
def kernel(hidden_states, router_logits, w13, w2):
    pos1, pos2, wt1, be, act, xi = _router(router_logits)
    pos1 = pos1.reshape(T)
    pos2 = pos2.reshape(T)
    be = be.reshape(NBP)
    act = act.reshape(NBP)
    xi = xi.reshape(NBP)
    disp = _dispatch(hidden_states, pos1, pos2)
    g = _gateup(disp, w13, be, act, xi)
    y = _down(g, w2, be, act, xi)
    return _combine(y, pos1, pos2, wt1)


# speedup vs baseline: 1.0081x; 1.0011x over previous
import jax
import jax.numpy as jnp
from jax import lax
from jax.experimental import pallas as pl
from jax.experimental.pallas import tpu as pltpu
from jax.experimental.pallas import tpu_sc as plsc

T, D, F, E = 2048, 1024, 2048, 8
BLK = 256
CAP = 2 * T + E * BLK
NB = CAP // BLK
NBP = 32
NW = 32
TPW = T // NW


def _router_kernel(logits_ref, pos1_ref, pos2_ref, wt1_ref, be_ref, act_ref,
                   xi_ref):
    l = logits_ref[...]
    ids = lax.broadcasted_iota(jnp.int32, (T, E), 1)
    m1 = jnp.max(l, axis=1, keepdims=True)
    i1 = jnp.argmax(l, axis=1)[:, None]
    masked = jnp.where(ids == i1, -jnp.inf, l)
    m2 = jnp.max(masked, axis=1, keepdims=True)
    i2 = jnp.argmax(masked, axis=1)[:, None]
    w1 = 1.0 / (1.0 + jnp.exp(m2 - m1))
    wt1_ref[...] = jnp.broadcast_to(w1, (T, 16))

    match = ((ids == i1) | (ids == i2)).astype(jnp.bfloat16)
    r = lax.broadcasted_iota(jnp.int32, (T, T), 0)
    c = lax.broadcasted_iota(jnp.int32, (T, T), 1)
    tri = (c < r).astype(jnp.bfloat16)
    rank = lax.dot_general(tri, match, (((1,), (0,)), ((), ())),
                           preferred_element_type=jnp.float32)
    counts = jnp.sum(match.astype(jnp.float32), axis=0)
    cnt = counts.astype(jnp.int32)

    pos1 = jnp.zeros((T, 1), jnp.int32)
    pos2 = jnp.zeros((T, 1), jnp.int32)
    start = jnp.int32(0)
    starts = []
    for e in range(E):
        starts.append(start)
        start = start + ((cnt[e] + BLK - 1) // BLK) * BLK
    for e in range(E):
        pe = starts[e] + rank[:, e:e + 1].astype(jnp.int32)
        pos1 = jnp.where(i1 == e, pe, pos1)
        pos2 = jnp.where(i2 == e, pe, pos2)
    pos1_ref[...] = pos1
    pos2_ref[...] = pos2

    blk_base = lax.broadcasted_iota(jnp.int32, (NBP, 1), 0) * BLK
    be = jnp.zeros((NBP, 1), jnp.int32)
    for e in range(1, E):
        be = be + (blk_base >= starts[e]).astype(jnp.int32)
    act = (blk_base < start).astype(jnp.int32)
    be_last = jnp.max(be * act)
    be_ref[...] = jnp.where(act > 0, be, be_last)
    act_ref[...] = act
    nact = (start + BLK - 1) // BLK
    blk_i = lax.broadcasted_iota(jnp.int32, (NBP, 1), 0)
    xi_ref[...] = jnp.minimum(blk_i, jnp.maximum(nact - 1, 0))


def _router(router_logits):
    return pl.pallas_call(
        _router_kernel,
        out_shape=[
            jax.ShapeDtypeStruct((T, 1), jnp.int32),
            jax.ShapeDtypeStruct((T, 1), jnp.int32),
            jax.ShapeDtypeStruct((T, 16), jnp.float32),
            jax.ShapeDtypeStruct((NBP, 1), jnp.int32),
            jax.ShapeDtypeStruct((NBP, 1), jnp.int32),
            jax.ShapeDtypeStruct((NBP, 1), jnp.int32),
        ],
    )(router_logits)


def _dispatch_body(x_hbm, pos1_hbm, pos2_hbm, disp_hbm,
                   rows_v, p1_v, p2_v, sem0, sem1):
    wid = lax.axis_index("s") * 2 + lax.axis_index("c")
    base = wid * TPW
    pltpu.sync_copy(x_hbm.at[pl.ds(base, TPW)], rows_v)
    pltpu.sync_copy(pos1_hbm.at[pl.ds(base, TPW)], p1_v)
    pltpu.sync_copy(pos2_hbm.at[pl.ds(base, TPW)], p2_v)
    c1 = pltpu.make_async_copy(rows_v, disp_hbm.at[p1_v], sem0)
    c1.start()
    c2 = pltpu.make_async_copy(rows_v, disp_hbm.at[p2_v], sem1)
    c2.start()
    c1.wait()
    c2.wait()


def _dispatch(hidden_states, pos1, pos2):
    mesh = plsc.VectorSubcoreMesh(core_axis_name="c", subcore_axis_name="s", num_cores=2, num_subcores=16)
    return pl.kernel(
        _dispatch_body,
        out_type=jax.ShapeDtypeStruct((CAP, D), jnp.float32),
        mesh=mesh,
        scratch_types=[
            pltpu.VMEM((TPW, D), jnp.float32),
            pltpu.VMEM((TPW,), jnp.int32),
            pltpu.VMEM((TPW,), jnp.int32),
            pltpu.SemaphoreType.DMA,
            pltpu.SemaphoreType.DMA,
        ],
    )(hidden_states, pos1, pos2)


def _gateup_kernel(be_sref, act_sref, xi_sref, x_ref, w13_ref, g_ref):
    del be_sref, xi_sref

    @pl.when(act_sref[pl.program_id(0)] > 0)
    def _():
        x = x_ref[...].astype(jnp.bfloat16)
        xT = x.T
        w13 = w13_ref[0]
        hT = lax.dot_general(w13, xT, (((1,), (0,)), ((), ())),
                             preferred_element_type=jnp.float32)
        hg = hT[:F, :]
        hu = hT[F:, :]
        g_ref[...] = (hg * jax.nn.sigmoid(hg) * hu).astype(jnp.bfloat16)


def _gateup(disp, w13, be, act, xi):
    grid_spec = pltpu.PrefetchScalarGridSpec(
        num_scalar_prefetch=3,
        grid=(NB,),
        in_specs=[
            pl.BlockSpec((BLK, D), lambda i, be, act, xi: (xi[i], 0)),
            pl.BlockSpec((1, 2 * F, D),
                         lambda i, be, act, xi: (be[i], 0, 0)),
        ],
        out_specs=pl.BlockSpec((F, BLK), lambda i, be, act, xi: (0, xi[i])),
    )
    return pl.pallas_call(
        _gateup_kernel,
        grid_spec=grid_spec,
        out_shape=jax.ShapeDtypeStruct((F, CAP), jnp.bfloat16),
    )(be, act, xi, disp, w13)


def _down_kernel(be_sref, act_sref, xi_sref, g_ref, w2_ref, y_ref):
    del be_sref, xi_sref

    @pl.when(act_sref[pl.program_id(0)] > 0)
    def _():
        g = g_ref[...]
        w2 = w2_ref[0]
        yT = lax.dot_general(w2, g, (((1,), (0,)), ((), ())),
                             preferred_element_type=jnp.float32)
        y_ref[...] = yT.T


def _down(g, w2, be, act, xi):
    grid_spec = pltpu.PrefetchScalarGridSpec(
        num_scalar_prefetch=3,
        grid=(NB,),
        in_specs=[
            pl.BlockSpec((F, BLK), lambda i, be, act, xi: (0, xi[i])),
            pl.BlockSpec((1, D, F), lambda i, be, act, xi: (be[i], 0, 0)),
        ],
        out_specs=pl.BlockSpec((BLK, D),
                               lambda i, be, act, xi: (xi[i], 0)),
    )
    return pl.pallas_call(
        _down_kernel,
        grid_spec=grid_spec,
        out_shape=jax.ShapeDtypeStruct((CAP, D), jnp.float32),
    )(be, act, xi, g, w2)


_CCH = 32


def _combine_body(y_hbm, pos1_hbm, pos2_hbm, wt1_hbm, out_hbm,
                  r1_v, r2_v, p1_v, p2_v, w1_v, sem, sem2):
    wid = lax.axis_index("s") * 2 + lax.axis_index("c")
    base = wid * TPW
    for chunk in range(TPW // _CCH):
        cbase = base + chunk * _CCH
        pltpu.sync_copy(pos1_hbm.at[pl.ds(cbase, _CCH)], p1_v)
        pltpu.sync_copy(pos2_hbm.at[pl.ds(cbase, _CCH)], p2_v)
        pltpu.sync_copy(wt1_hbm.at[pl.ds(cbase, _CCH)], w1_v)
        c1 = pltpu.make_async_copy(y_hbm.at[p1_v], r1_v, sem)
        c1.start()
        c2 = pltpu.make_async_copy(y_hbm.at[p2_v], r2_v, sem2)
        c2.start()
        c1.wait()
        c2.wait()

        def body(t, carry):
            w1 = w1_v[t, :]
            w2 = 1.0 - w1
            for j in range(D // 16):
                sl = pl.ds(j * 16, 16)
                r1_v[t, sl] = w1 * r1_v[t, sl] + w2 * r2_v[t, sl]
            return carry

        lax.fori_loop(0, _CCH, body, jnp.int32(0))
        pltpu.sync_copy(r1_v, out_hbm.at[pl.ds(cbase, _CCH)])


def _combine(y, pos1, pos2, wt1):
    mesh = plsc.VectorSubcoreMesh(core_axis_name="c", subcore_axis_name="s", num_cores=2, num_subcores=16)
    return pl.kernel(
        _combine_body,
        out_type=jax.ShapeDtypeStruct((T, D), jnp.float32),
        mesh=mesh,
        scratch_types=[
            pltpu.VMEM((_CCH, D), jnp.float32),
            pltpu.VMEM((_CCH, D), jnp.float32),
            pltpu.VMEM((_CCH,), jnp.int32),
            pltpu.VMEM((_CCH,), jnp.int32),
            pltpu.VMEM((_CCH, 16), jnp.float32),
            pltpu.SemaphoreType.DMA,
            pltpu.SemaphoreType.DMA,
        ],
    )(y, pos1, pos2, wt1)


@jax.jit
def kernel(hidden_states, router_logits, w13, w2):
    pos1, pos2, wt1, be, act, xi = _router(router_logits)
    pos1 = pos1.reshape(T)
    pos2 = pos2.reshape(T)
    be = be.reshape(NBP)
    act = act.reshape(NBP)
    xi = xi.reshape(NBP)
    disp = _dispatch(hidden_states, pos1, pos2)
    g = _gateup(disp, w13, be, act, xi)
    y = _down(g, w2, be, act, xi)
    return _combine(y, pos1, pos2, wt1)
